# baseline (speedup 1.0000x reference)
import jax
import jax.numpy as jnp
from jax import lax
from jax.experimental import pallas as pl
from jax.experimental.pallas import tpu as pltpu

_RB = 512


def _gcn_body(a_ref, x_ref, w_ref, b_ref, o_ref, abf_ref, d_ref, ybf_ref, acc_ref):
    k = pl.program_id(0)
    ns = abf_ref.shape[0]

    @pl.when(k < ns)
    def _load_stripe():
        a = a_ref[...]
        a_bf = a.astype(jnp.bfloat16)
        for j in range(ns):
            abf_ref[pl.ds(k, 1), j, :, :] = a_bf[None, :, j * _RB:(j + 1) * _RB]
        deg = jnp.sum(a, axis=1, keepdims=True) + 1.0
        d = lax.rsqrt(deg)
        d_ref[pl.ds(k, 1)] = d[None]
        xw = lax.dot_general(
            x_ref[...], w_ref[...],
            dimension_numbers=(((1,), (1,)), ((), ())),
            preferred_element_type=jnp.float32,
        )
        ybf_ref[pl.ds(k, 1)] = (d * xw).astype(jnp.bfloat16)[None]
        acc_ref[pl.ds(k * _RB, _RB), :] = jnp.zeros((_RB, acc_ref.shape[1]),
                                                    jnp.float32)

    @pl.when(k > 0)
    def _wavefront():
        c = k - 1
        yc = ybf_ref[pl.ds(c, 1)][0]

        def row_blocks(i, _):
            blk = abf_ref[pl.ds(i, 1), c, :, :][0]
            z = lax.dot_general(
                blk, yc,
                dimension_numbers=(((1,), (0,)), ((), ())),
                preferred_element_type=jnp.float32,
            )
            acc_ref[pl.ds(i * _RB, _RB), :] += z
            return 0

        lax.fori_loop(0, k, row_blocks, 0)

        def col_blocks(j, _):
            blk = abf_ref[pl.ds(c, 1), j, :, :][0]
            yj = ybf_ref[pl.ds(j, 1)][0]
            z = lax.dot_general(
                blk, yj,
                dimension_numbers=(((1,), (0,)), ((), ())),
                preferred_element_type=jnp.float32,
            )
            acc_ref[pl.ds(c * _RB, _RB), :] += z
            return 0

        lax.fori_loop(0, k - 1, col_blocks, 0)

    @pl.when(k == ns)
    def _epilogue():
        for i in range(ns):
            d = d_ref[pl.ds(i, 1)][0]
            y = ybf_ref[pl.ds(i, 1)][0].astype(jnp.float32)
            acc = acc_ref[pl.ds(i * _RB, _RB), :]
            o_ref[pl.ds(i * _RB, _RB), :] = d * acc + d * y + b_ref[...]


def kernel(x, A, W, b):
    n, din = x.shape
    dout = W.shape[0]
    ns = n // _RB

    out = pl.pallas_call(
        _gcn_body,
        grid=(ns + 1,),
        in_specs=[
            pl.BlockSpec((_RB, n), lambda k: (jnp.minimum(k, ns - 1), 0)),
            pl.BlockSpec((_RB, din), lambda k: (jnp.minimum(k, ns - 1), 0)),
            pl.BlockSpec((dout, din), lambda k: (0, 0)),
            pl.BlockSpec((1, dout), lambda k: (0, 0)),
        ],
        out_specs=pl.BlockSpec((n, dout), lambda k: (0, 0)),
        out_shape=jax.ShapeDtypeStruct((n, dout), jnp.float32),
        scratch_shapes=[
            pltpu.VMEM((ns, ns, _RB, _RB), jnp.bfloat16),
            pltpu.VMEM((ns, _RB, 1), jnp.float32),
            pltpu.VMEM((ns, _RB, dout), jnp.bfloat16),
            pltpu.VMEM((n, dout), jnp.float32),
        ],
    )(A, x, W, b.reshape(1, dout))
    return out
